# Initial kernel scaffold; baseline (speedup 1.0000x reference)
import jax
import jax.numpy as jnp
from jax.experimental import pallas as pl


def kernel(x, edge_index, edge_attr, W1, b1, W2, b2, Wg, bg, gamma, beta):
    raise NotImplementedError("write your pallas kernel here")


# speedup vs baseline: 2.7351x; 2.7351x over previous
import functools

import jax
import jax.numpy as jnp
from jax import lax
from jax.experimental import pallas as pl
from jax.experimental.pallas import tpu as pltpu
from jax.experimental.pallas import tpu_sc as plsc

NC, NS = 2, 16
NW = NC * NS
CHUNK = 128


def _sc_gather(x, row, E, N, D):
    n_chunks = E // CHUNK
    per_w = -(-n_chunks // NW)
    mesh = plsc.VectorSubcoreMesh(core_axis_name="c", subcore_axis_name="s")

    @functools.partial(
        pl.kernel,
        out_type=jax.ShapeDtypeStruct((E, D), jnp.float32),
        mesh=mesh,
        scratch_types=[
            pltpu.VMEM((CHUNK,), jnp.int32),
            pltpu.VMEM((CHUNK, D), jnp.float32),
            pltpu.SemaphoreType.DMA,
        ],
    )
    def k(x_hbm, row_hbm, out_hbm, idx_v, rows_v, sem):
        wid = lax.axis_index("s") * NC + lax.axis_index("c")

        def body(t, carry):
            chunk = wid + t * NW

            @pl.when(chunk < n_chunks)
            def _():
                base = chunk * CHUNK
                pltpu.sync_copy(row_hbm.at[pl.ds(base, CHUNK)], idx_v)
                pltpu.async_copy(x_hbm.at[idx_v], rows_v, sem).wait()
                pltpu.sync_copy(rows_v, out_hbm.at[pl.ds(base, CHUNK)])

            return carry

        lax.fori_loop(0, per_w, body, 0)

    return k(x, row)


def _sc_scatter(msg, col, zeros, E, N, D):
    n_chunks = E // CHUNK
    per_w = -(-n_chunks // NW)
    SLAB = (N // NS) // 8 * 8
    LAST = N - (NS - 1) * SLAB
    mesh = plsc.VectorSubcoreMesh(core_axis_name="c", subcore_axis_name="s")

    @functools.partial(
        pl.kernel,
        out_type=jax.ShapeDtypeStruct((NC, N, D), jnp.float32),
        mesh=mesh,
        scratch_types=[
            pltpu.VMEM((CHUNK,), jnp.int32),
            pltpu.VMEM((CHUNK, D), jnp.float32),
            pltpu.VMEM_SHARED((N, D), jnp.float32),
        ],
    )
    def k(msg_hbm, col_hbm, zero_hbm, parts_hbm, idx_v, upd_v, acc_sh):
        c = lax.axis_index("c")
        s = lax.axis_index("s")
        wid = s * NC + c
        r0 = s * SLAB

        @pl.when(s < NS - 1)
        def _():
            pltpu.sync_copy(
                zero_hbm.at[pl.ds(r0, SLAB)], acc_sh.at[pl.ds(r0, SLAB)]
            )

        @pl.when(s == NS - 1)
        def _():
            pltpu.sync_copy(
                zero_hbm.at[pl.ds(r0, LAST)], acc_sh.at[pl.ds(r0, LAST)]
            )

        plsc.subcore_barrier()

        def body(t, carry):
            chunk = wid + t * NW

            @pl.when(chunk < n_chunks)
            def _():
                base = chunk * CHUNK
                pltpu.sync_copy(col_hbm.at[pl.ds(base, CHUNK)], idx_v)
                pltpu.sync_copy(msg_hbm.at[pl.ds(base, CHUNK)], upd_v)
                pltpu.sync_copy(upd_v, acc_sh.at[idx_v], add=True)

            return carry

        lax.fori_loop(0, per_w, body, 0)
        plsc.subcore_barrier()

        @pl.when(s < NS - 1)
        def _():
            pltpu.sync_copy(
                acc_sh.at[pl.ds(r0, SLAB)],
                parts_hbm.at[c].at[pl.ds(r0, SLAB)],
            )

        @pl.when(s == NS - 1)
        def _():
            pltpu.sync_copy(
                acc_sh.at[pl.ds(r0, LAST)],
                parts_hbm.at[c].at[pl.ds(r0, LAST)],
            )

    return k(msg, col, zeros)


def _tc_mlp(src, ea, W1a, W1b, b1, W2, b2, Wga, Wgb, bg, E, D, ED):
    BLK = 1280
    grid = (E // BLK,)

    def body(src_r, ea_r, W1a_r, W1b_r, b1_r, W2_r, b2_r, Wga_r, Wgb_r, bg_r, out_r):
        s = src_r[...]
        a = ea_r[...]
        h = s @ W1a_r[...] + a @ W1b_r[...] + b1_r[...]
        h = jnp.maximum(h, 0.0)
        core = h @ W2_r[...] + b2_r[...]
        g = s @ Wga_r[...] + a @ Wgb_r[...] + bg_r[...]
        gate = 1.0 / (1.0 + jnp.exp(-g))
        out_r[...] = gate * core

    full = lambda shape: pl.BlockSpec(shape, lambda i: (0, 0))
    return pl.pallas_call(
        body,
        grid=grid,
        in_specs=[
            pl.BlockSpec((BLK, D), lambda i: (i, 0)),
            pl.BlockSpec((BLK, ED), lambda i: (i, 0)),
            full((D, D)),
            full((ED, D)),
            full((1, D)),
            full((D, D)),
            full((1, D)),
            full((D, D)),
            full((ED, D)),
            full((1, D)),
        ],
        out_specs=pl.BlockSpec((BLK, D), lambda i: (i, 0)),
        out_shape=jax.ShapeDtypeStruct((E, D), jnp.float32),
    )(src, ea, W1a, W1b, b1, W2, b2, Wga, Wgb, bg)


def _tc_ln(p0, p1, x, gamma, beta, N, D):
    BLK = 1000
    grid = (N // BLK,)

    def body(p0_r, p1_r, x_r, g_r, b_r, out_r):
        r = p0_r[...] + p1_r[...] + x_r[...]
        m = jnp.mean(r, axis=1, keepdims=True)
        d = r - m
        v = jnp.mean(d * d, axis=1, keepdims=True)
        out_r[...] = d * lax.rsqrt(v + 1e-5) * g_r[...] + b_r[...]

    return pl.pallas_call(
        body,
        grid=grid,
        in_specs=[
            pl.BlockSpec((BLK, D), lambda i: (i, 0)),
            pl.BlockSpec((BLK, D), lambda i: (i, 0)),
            pl.BlockSpec((BLK, D), lambda i: (i, 0)),
            pl.BlockSpec((1, D), lambda i: (0, 0)),
            pl.BlockSpec((1, D), lambda i: (0, 0)),
        ],
        out_specs=pl.BlockSpec((BLK, D), lambda i: (i, 0)),
        out_shape=jax.ShapeDtypeStruct((N, D), jnp.float32),
    )(p0, p1, x, gamma, beta)


def kernel(x, edge_index, edge_attr, W1, b1, W2, b2, Wg, bg, gamma, beta):
    N, D = x.shape
    E = edge_index.shape[1]
    ED = edge_attr.shape[1]

    row = edge_index[0]
    col = edge_index[1]
    W1a, W1b = W1[:D], W1[D:]
    Wga, Wgb = Wg[:D], Wg[D:]
    b1r = b1.reshape(1, D)
    b2r = b2.reshape(1, D)
    bgr = bg.reshape(1, D)
    zeros = jnp.zeros((N, D), jnp.float32)

    src = _sc_gather(x, row, E, N, D)
    msg = _tc_mlp(src, edge_attr, W1a, W1b, b1r, W2, b2r, Wga, Wgb, bgr, E, D, ED)
    parts = _sc_scatter(msg, col, zeros, E, N, D)
    out = _tc_ln(parts[0], parts[1], x, gamma.reshape(1, D), beta.reshape(1, D), N, D)
    return out


# speedup vs baseline: 3.3902x; 1.2395x over previous
import functools

import jax
import jax.numpy as jnp
from jax import lax
from jax.experimental import pallas as pl
from jax.experimental.pallas import tpu as pltpu
from jax.experimental.pallas import tpu_sc as plsc

NC, NS = 2, 16
NW = NC * NS
CHUNK = 128


def _sc_gather(x, row, E, N, D):
    PW = E // NW
    SCH = 200
    NSCH = PW // SCH
    SPLITS = [(0, 128), (128, SCH - 128)]
    mesh = plsc.VectorSubcoreMesh(core_axis_name="c", subcore_axis_name="s")

    @functools.partial(
        pl.kernel,
        out_type=jax.ShapeDtypeStruct((E, D), jnp.float32),
        mesh=mesh,
        scratch_types=[
            pltpu.VMEM((PW,), jnp.int32),
            pltpu.VMEM((SCH, D), jnp.float32),
            pltpu.VMEM((SCH, D), jnp.float32),
            pltpu.SemaphoreType.DMA,
            pltpu.SemaphoreType.DMA,
            pltpu.SemaphoreType.DMA,
            pltpu.SemaphoreType.DMA,
        ],
    )
    def k(x_hbm, row_hbm, out_hbm, idx_all, rows0, rows1, g0, g1, s0, s1):
        wid = lax.axis_index("s") * NC + lax.axis_index("c")
        e0 = wid * PW
        rows = (rows0, rows1)
        gsem = (g0, g1)
        ssem = (s0, s1)

        pltpu.sync_copy(row_hbm.at[pl.ds(e0, PW)], idx_all)

        def fire_gathers(j, b):
            for (off, ln) in SPLITS:
                pltpu.async_copy(
                    x_hbm.at[idx_all.at[pl.ds(j * SCH + off, ln)]],
                    rows[b].at[pl.ds(off, ln)],
                    gsem[b],
                )

        def wait_gathers(j, b):
            for (off, ln) in SPLITS:
                pltpu.make_async_copy(
                    x_hbm.at[idx_all.at[pl.ds(j * SCH + off, ln)]],
                    rows[b].at[pl.ds(off, ln)],
                    gsem[b],
                ).wait()

        def store_desc(j, b):
            return pltpu.make_async_copy(
                rows[b], out_hbm.at[pl.ds(e0 + j * SCH, SCH)], ssem[b]
            )

        fire_gathers(0, 0)

        def body(j, carry):
            nb = 1 - (j % 2)
            for b in (0, 1):

                @pl.when(j % 2 == b)
                def _():
                    wait_gathers(j, b)
                    store_desc(j, b).start()

                @pl.when((j % 2 != b) & (j + 1 < NSCH))
                def _():
                    @pl.when(j >= 1)
                    def _():
                        store_desc(j - 1, b).wait()

                    fire_gathers(j + 1, b)

            return carry

        lax.fori_loop(0, NSCH, body, 0, unroll=2)
        store_desc(NSCH - 2, (NSCH - 2) % 2).wait()
        store_desc(NSCH - 1, (NSCH - 1) % 2).wait()

    return k(x, row)


def _sc_scatter(msg, col, zeros, E, N, D):
    n_chunks = E // CHUNK
    per_w = -(-n_chunks // NW)
    SLAB = (N // NS) // 8 * 8
    LAST = N - (NS - 1) * SLAB
    mesh = plsc.VectorSubcoreMesh(core_axis_name="c", subcore_axis_name="s")

    @functools.partial(
        pl.kernel,
        out_type=jax.ShapeDtypeStruct((NC, N, D), jnp.float32),
        mesh=mesh,
        scratch_types=[
            pltpu.VMEM((CHUNK,), jnp.int32),
            pltpu.VMEM((CHUNK,), jnp.int32),
            pltpu.VMEM((CHUNK, D), jnp.float32),
            pltpu.VMEM((CHUNK, D), jnp.float32),
            pltpu.SemaphoreType.DMA,
            pltpu.SemaphoreType.DMA,
            pltpu.VMEM_SHARED((N, D), jnp.float32),
        ],
    )
    def k(msg_hbm, col_hbm, zero_hbm, parts_hbm, idx0, idx1, upd0, upd1,
          i0, i1, acc_sh):
        c = lax.axis_index("c")
        s = lax.axis_index("s")
        wid = s * NC + c
        r0 = s * SLAB
        idx_v = (idx0, idx1)
        upd_v = (upd0, upd1)
        isem = (i0, i1)

        @pl.when(s < NS - 1)
        def _():
            pltpu.sync_copy(
                zero_hbm.at[pl.ds(r0, SLAB)], acc_sh.at[pl.ds(r0, SLAB)]
            )

        @pl.when(s == NS - 1)
        def _():
            pltpu.sync_copy(
                zero_hbm.at[pl.ds(r0, LAST)], acc_sh.at[pl.ds(r0, LAST)]
            )

        plsc.subcore_barrier()

        def in_descs(t, b):
            base = (wid + t * NW) * CHUNK
            return (
                pltpu.make_async_copy(
                    col_hbm.at[pl.ds(base, CHUNK)], idx_v[b], isem[b]
                ),
                pltpu.make_async_copy(
                    msg_hbm.at[pl.ds(base, CHUNK)], upd_v[b], isem[b]
                ),
            )

        def fire_in(t, b):
            @pl.when(wid + t * NW < n_chunks)
            def _():
                for d in in_descs(t, b):
                    d.start()

        fire_in(0, 0)
        fire_in(1, 1)

        def body(t, carry):
            for b in (0, 1):

                @pl.when((t % 2 == b) & (wid + t * NW < n_chunks))
                def _():
                    for d in in_descs(t, b):
                        d.wait()
                    pltpu.sync_copy(upd_v[b], acc_sh.at[idx_v[b]], add=True)
                    fire_in(t + 2, b)

            return carry

        lax.fori_loop(0, per_w, body, 0, unroll=2)
        plsc.subcore_barrier()

        @pl.when(s < NS - 1)
        def _():
            pltpu.sync_copy(
                acc_sh.at[pl.ds(r0, SLAB)],
                parts_hbm.at[c].at[pl.ds(r0, SLAB)],
            )

        @pl.when(s == NS - 1)
        def _():
            pltpu.sync_copy(
                acc_sh.at[pl.ds(r0, LAST)],
                parts_hbm.at[c].at[pl.ds(r0, LAST)],
            )

    return k(msg, col, zeros)


def _tc_mlp(src, ea, W1a, W1b, b1, W2, b2, Wga, Wgb, bg, E, D, ED):
    BLK = 1280
    grid = (E // BLK,)

    def body(src_r, ea_r, W1a_r, W1b_r, b1_r, W2_r, b2_r, Wga_r, Wgb_r, bg_r, out_r):
        s = src_r[...]
        a = ea_r[...]
        h = s @ W1a_r[...] + a @ W1b_r[...] + b1_r[...]
        h = jnp.maximum(h, 0.0)
        core = h @ W2_r[...] + b2_r[...]
        g = s @ Wga_r[...] + a @ Wgb_r[...] + bg_r[...]
        gate = 1.0 / (1.0 + jnp.exp(-g))
        out_r[...] = gate * core

    full = lambda shape: pl.BlockSpec(shape, lambda i: (0, 0))
    return pl.pallas_call(
        body,
        grid=grid,
        in_specs=[
            pl.BlockSpec((BLK, D), lambda i: (i, 0)),
            pl.BlockSpec((BLK, ED), lambda i: (i, 0)),
            full((D, D)),
            full((ED, D)),
            full((1, D)),
            full((D, D)),
            full((1, D)),
            full((D, D)),
            full((ED, D)),
            full((1, D)),
        ],
        out_specs=pl.BlockSpec((BLK, D), lambda i: (i, 0)),
        out_shape=jax.ShapeDtypeStruct((E, D), jnp.float32),
    )(src, ea, W1a, W1b, b1, W2, b2, Wga, Wgb, bg)


def _tc_ln(p0, p1, x, gamma, beta, N, D):
    BLK = 1000
    grid = (N // BLK,)

    def body(p0_r, p1_r, x_r, g_r, b_r, out_r):
        r = p0_r[...] + p1_r[...] + x_r[...]
        m = jnp.mean(r, axis=1, keepdims=True)
        d = r - m
        v = jnp.mean(d * d, axis=1, keepdims=True)
        out_r[...] = d * lax.rsqrt(v + 1e-5) * g_r[...] + b_r[...]

    return pl.pallas_call(
        body,
        grid=grid,
        in_specs=[
            pl.BlockSpec((BLK, D), lambda i: (i, 0)),
            pl.BlockSpec((BLK, D), lambda i: (i, 0)),
            pl.BlockSpec((BLK, D), lambda i: (i, 0)),
            pl.BlockSpec((1, D), lambda i: (0, 0)),
            pl.BlockSpec((1, D), lambda i: (0, 0)),
        ],
        out_specs=pl.BlockSpec((BLK, D), lambda i: (i, 0)),
        out_shape=jax.ShapeDtypeStruct((N, D), jnp.float32),
    )(p0, p1, x, gamma, beta)


def kernel(x, edge_index, edge_attr, W1, b1, W2, b2, Wg, bg, gamma, beta):
    N, D = x.shape
    E = edge_index.shape[1]
    ED = edge_attr.shape[1]

    row = edge_index[0]
    col = edge_index[1]
    W1a, W1b = W1[:D], W1[D:]
    Wga, Wgb = Wg[:D], Wg[D:]
    b1r = b1.reshape(1, D)
    b2r = b2.reshape(1, D)
    bgr = bg.reshape(1, D)
    zeros = jnp.zeros((N, D), jnp.float32)

    src = _sc_gather(x, row, E, N, D)
    msg = _tc_mlp(src, edge_attr, W1a, W1b, b1r, W2, b2r, Wga, Wgb, bgr, E, D, ED)
    parts = _sc_scatter(msg, col, zeros, E, N, D)
    out = _tc_ln(parts[0], parts[1], x, gamma.reshape(1, D), beta.reshape(1, D), N, D)
    return out


# speedup vs baseline: 3.5774x; 1.0552x over previous
import functools

import jax
import jax.numpy as jnp
from jax import lax
from jax.experimental import pallas as pl
from jax.experimental.pallas import tpu as pltpu
from jax.experimental.pallas import tpu_sc as plsc

NC, NS = 2, 16
NW = NC * NS
CHUNK = 128


def _sc_gather(x, row, E, N, D):
    PW = E // NW
    SCH = 200
    NSCH = PW // SCH
    SPLITS = [(0, 128), (128, SCH - 128)]
    mesh = plsc.VectorSubcoreMesh(core_axis_name="c", subcore_axis_name="s")

    @functools.partial(
        pl.kernel,
        out_type=jax.ShapeDtypeStruct((E, D), jnp.float32),
        mesh=mesh,
        scratch_types=[
            pltpu.VMEM((PW,), jnp.int32),
            pltpu.VMEM((SCH, D), jnp.float32),
            pltpu.VMEM((SCH, D), jnp.float32),
            pltpu.SemaphoreType.DMA,
            pltpu.SemaphoreType.DMA,
            pltpu.SemaphoreType.DMA,
            pltpu.SemaphoreType.DMA,
        ],
    )
    def k(x_hbm, row_hbm, out_hbm, idx_all, rows0, rows1, g0, g1, s0, s1):
        wid = lax.axis_index("s") * NC + lax.axis_index("c")
        e0 = wid * PW
        rows = (rows0, rows1)
        gsem = (g0, g1)
        ssem = (s0, s1)

        pltpu.sync_copy(row_hbm.at[pl.ds(e0, PW)], idx_all)

        def fire_gathers(j, b):
            for (off, ln) in SPLITS:
                pltpu.async_copy(
                    x_hbm.at[idx_all.at[pl.ds(j * SCH + off, ln)]],
                    rows[b].at[pl.ds(off, ln)],
                    gsem[b],
                )

        def wait_gathers(j, b):
            for (off, ln) in SPLITS:
                pltpu.make_async_copy(
                    x_hbm.at[idx_all.at[pl.ds(j * SCH + off, ln)]],
                    rows[b].at[pl.ds(off, ln)],
                    gsem[b],
                ).wait()

        def store_desc(j, b):
            return pltpu.make_async_copy(
                rows[b], out_hbm.at[pl.ds(e0 + j * SCH, SCH)], ssem[b]
            )

        fire_gathers(0, 0)

        def body(j, carry):
            nb = 1 - (j % 2)
            for b in (0, 1):

                @pl.when(j % 2 == b)
                def _():
                    wait_gathers(j, b)
                    store_desc(j, b).start()

                @pl.when((j % 2 != b) & (j + 1 < NSCH))
                def _():
                    @pl.when(j >= 1)
                    def _():
                        store_desc(j - 1, b).wait()

                    fire_gathers(j + 1, b)

            return carry

        lax.fori_loop(0, NSCH, body, 0, unroll=2)
        store_desc(NSCH - 2, (NSCH - 2) % 2).wait()
        store_desc(NSCH - 1, (NSCH - 1) % 2).wait()

    return k(x, row)


def _sc_scatter(msgs, cols, zeros, E, N, D):
    K = len(msgs)
    Eh = E // K
    n_chunks = Eh // CHUNK
    per_w = -(-n_chunks // NW)
    SLAB = (N // NS) // 8 * 8
    LAST = N - (NS - 1) * SLAB
    mesh = plsc.VectorSubcoreMesh(core_axis_name="c", subcore_axis_name="s")

    @functools.partial(
        pl.kernel,
        out_type=jax.ShapeDtypeStruct((NC, N, D), jnp.float32),
        mesh=mesh,
        scratch_types=[
            pltpu.VMEM((CHUNK,), jnp.int32),
            pltpu.VMEM((CHUNK,), jnp.int32),
            pltpu.VMEM((CHUNK, D), jnp.float32),
            pltpu.VMEM((CHUNK, D), jnp.float32),
            pltpu.SemaphoreType.DMA,
            pltpu.SemaphoreType.DMA,
            pltpu.VMEM_SHARED((N, D), jnp.float32),
        ],
    )
    def k(*args):
        msg_hbm = args[:K]
        col_hbm = args[K:2 * K]
        zero_hbm = args[2 * K]
        parts_hbm = args[2 * K + 1]
        idx0, idx1, upd0, upd1, i0, i1, acc_sh = args[2 * K + 2:]
        c = lax.axis_index("c")
        s = lax.axis_index("s")
        wid = s * NC + c
        r0 = s * SLAB
        idx_v = (idx0, idx1)
        upd_v = (upd0, upd1)
        isem = (i0, i1)

        @pl.when(s < NS - 1)
        def _():
            pltpu.sync_copy(
                zero_hbm.at[pl.ds(r0, SLAB)], acc_sh.at[pl.ds(r0, SLAB)]
            )

        @pl.when(s == NS - 1)
        def _():
            pltpu.sync_copy(
                zero_hbm.at[pl.ds(r0, LAST)], acc_sh.at[pl.ds(r0, LAST)]
            )

        plsc.subcore_barrier()

        for h in range(K):

            def in_descs(t, b, h=h):
                base = (wid + t * NW) * CHUNK
                return (
                    pltpu.make_async_copy(
                        col_hbm[h].at[pl.ds(base, CHUNK)], idx_v[b], isem[b]
                    ),
                    pltpu.make_async_copy(
                        msg_hbm[h].at[pl.ds(base, CHUNK)], upd_v[b], isem[b]
                    ),
                )

            def fire_in(t, b, h=h):
                @pl.when(wid + t * NW < n_chunks)
                def _():
                    for d in in_descs(t, b, h):
                        d.start()

            fire_in(0, 0)
            fire_in(1, 1)

            def body(t, carry, in_descs=in_descs, fire_in=fire_in):
                for b in (0, 1):

                    @pl.when((t % 2 == b) & (wid + t * NW < n_chunks))
                    def _():
                        for d in in_descs(t, b):
                            d.wait()
                        pltpu.sync_copy(upd_v[b], acc_sh.at[idx_v[b]], add=True)
                        fire_in(t + 2, b)

                return carry

            lax.fori_loop(0, per_w, body, 0, unroll=2)

        plsc.subcore_barrier()

        @pl.when(s < NS - 1)
        def _():
            pltpu.sync_copy(
                acc_sh.at[pl.ds(r0, SLAB)],
                parts_hbm.at[c].at[pl.ds(r0, SLAB)],
            )

        @pl.when(s == NS - 1)
        def _():
            pltpu.sync_copy(
                acc_sh.at[pl.ds(r0, LAST)],
                parts_hbm.at[c].at[pl.ds(r0, LAST)],
            )

    return k(*msgs, *cols, zeros)


def _tc_mlp(src, ea, W1a, W1b, b1, W2, b2, Wga, Wgb, bg, E, D, ED):
    BLK = 2000
    grid = (E // BLK,)

    def body(src_r, ea_r, W1a_r, W1b_r, b1_r, W2_r, b2_r, Wga_r, Wgb_r, bg_r, out_r):
        s = src_r[...]
        a = ea_r[...]
        h = s @ W1a_r[...] + a @ W1b_r[...] + b1_r[...]
        h = jnp.maximum(h, 0.0)
        core = h @ W2_r[...] + b2_r[...]
        g = s @ Wga_r[...] + a @ Wgb_r[...] + bg_r[...]
        gate = 1.0 / (1.0 + jnp.exp(-g))
        out_r[...] = gate * core

    full = lambda shape: pl.BlockSpec(shape, lambda i: (0, 0))
    return pl.pallas_call(
        body,
        grid=grid,
        in_specs=[
            pl.BlockSpec((BLK, D), lambda i: (i, 0)),
            pl.BlockSpec((BLK, ED), lambda i: (i, 0)),
            full((D, D)),
            full((ED, D)),
            full((1, D)),
            full((D, D)),
            full((1, D)),
            full((D, D)),
            full((ED, D)),
            full((1, D)),
        ],
        out_specs=pl.BlockSpec((BLK, D), lambda i: (i, 0)),
        out_shape=jax.ShapeDtypeStruct((E, D), jnp.float32),
        compiler_params=pltpu.CompilerParams(
            dimension_semantics=("arbitrary",),
        ),
    )(src, ea, W1a, W1b, b1, W2, b2, Wga, Wgb, bg)


def _tc_ln(p0, p1, x, gamma, beta, N, D):
    BLK = 1000
    grid = (N // BLK,)

    def body(p0_r, p1_r, x_r, g_r, b_r, out_r):
        r = p0_r[...] + p1_r[...] + x_r[...]
        m = jnp.mean(r, axis=1, keepdims=True)
        d = r - m
        v = jnp.mean(d * d, axis=1, keepdims=True)
        out_r[...] = d * lax.rsqrt(v + 1e-5) * g_r[...] + b_r[...]

    return pl.pallas_call(
        body,
        grid=grid,
        in_specs=[
            pl.BlockSpec((BLK, D), lambda i: (i, 0)),
            pl.BlockSpec((BLK, D), lambda i: (i, 0)),
            pl.BlockSpec((BLK, D), lambda i: (i, 0)),
            pl.BlockSpec((1, D), lambda i: (0, 0)),
            pl.BlockSpec((1, D), lambda i: (0, 0)),
        ],
        out_specs=pl.BlockSpec((BLK, D), lambda i: (i, 0)),
        out_shape=jax.ShapeDtypeStruct((N, D), jnp.float32),
    )(p0, p1, x, gamma, beta)


def kernel(x, edge_index, edge_attr, W1, b1, W2, b2, Wg, bg, gamma, beta):
    N, D = x.shape
    E = edge_index.shape[1]
    ED = edge_attr.shape[1]

    row = edge_index[0]
    col = edge_index[1]
    W1a, W1b = W1[:D], W1[D:]
    Wga, Wgb = Wg[:D], Wg[D:]
    b1r = b1.reshape(1, D)
    b2r = b2.reshape(1, D)
    bgr = bg.reshape(1, D)
    zeros = jnp.zeros((N, D), jnp.float32)

    K = 2
    Eh = E // K
    msgs, cols = [], []
    for h in range(K):
        row_h = lax.dynamic_slice_in_dim(row, h * Eh, Eh)
        ea_h = lax.dynamic_slice_in_dim(edge_attr, h * Eh, Eh)
        src_h = _sc_gather(x, row_h, Eh, N, D)
        msgs.append(
            _tc_mlp(src_h, ea_h, W1a, W1b, b1r, W2, b2r, Wga, Wgb, bgr, Eh, D, ED)
        )
        cols.append(lax.dynamic_slice_in_dim(col, h * Eh, Eh))
    parts = _sc_scatter(msgs, cols, zeros, E, N, D)
    out = _tc_ln(parts[0], parts[1], x, gamma.reshape(1, D), beta.reshape(1, D), N, D)
    return out
